# initial kernel scaffold (unmeasured)
import functools

import jax
import jax.numpy as jnp
from jax import lax
from jax.experimental import pallas as pl
from jax.experimental.pallas import tpu as pltpu

M = 1024
N = 512


def kernel(x, dest):
    dest2d = dest.reshape(1, M)

    def body(x_ref, dest_ref, out_ref, xsend_ref, xpeer_ref, dpeer_ref,
             send_sems, recv_sems):
        my_x = lax.axis_index("x")
        my_y = lax.axis_index("y")
        peer = (1 - my_x, my_y)

        xsend_ref[...] = x_ref[...].astype(jnp.bfloat16)

        barrier_sem = pltpu.get_barrier_semaphore()
        pl.semaphore_signal(barrier_sem, inc=1, device_id=peer,
                            device_id_type=pl.DeviceIdType.MESH)
        pl.semaphore_wait(barrier_sem, 1)

        rdma_d = pltpu.make_async_remote_copy(
            src_ref=dest_ref, dst_ref=dpeer_ref,
            send_sem=send_sems.at[1], recv_sem=recv_sems.at[1],
            device_id=peer, device_id_type=pl.DeviceIdType.MESH)
        rdma_d.start()
        rdma_x = pltpu.make_async_remote_copy(
            src_ref=xsend_ref, dst_ref=xpeer_ref,
            send_sem=send_sems.at[0], recv_sem=recv_sems.at[0],
            device_id=peer, device_id_type=pl.DeviceIdType.MESH)
        rdma_x.start()
        rdma_d.wait()

        dl = dest_ref[...]
        dp = dpeer_ref[...]
        ml = (dl == my_x)
        mp = (dp == my_x)
        csl = jnp.cumsum(ml.astype(jnp.int32), axis=1)
        csp = jnp.cumsum(mp.astype(jnp.int32), axis=1)
        cl = csl[0, M - 1]
        cp = csp[0, M - 1]
        off_l = jnp.where(my_x == 0, 0, cp)
        off_p = jnp.where(my_x == 0, cl, 0)
        posl = off_l + csl - 1
        posp = off_p + csp - 1

        iota_j = lax.broadcasted_iota(jnp.int32, (M, M), 0)
        p_l = ((iota_j == posl) & ml).astype(jnp.bfloat16)
        acc = jnp.dot(p_l, xsend_ref[...], preferred_element_type=jnp.float32)

        rdma_x.wait()
        p_p = ((iota_j == posp) & mp).astype(jnp.bfloat16)
        acc = acc + jnp.dot(p_p, xpeer_ref[...],
                            preferred_element_type=jnp.float32)
        out_ref[...] = acc.astype(jnp.bfloat16)

        @functools.partial(pl.run_scoped, sem2=pltpu.SemaphoreType.REGULAR)
        def _(sem2):
            pl.semaphore_signal(sem2, inc=1, device_id=peer,
                                device_id_type=pl.DeviceIdType.MESH)
            pl.semaphore_wait(sem2, 1)

    return pl.pallas_call(
        body,
        out_shape=jax.ShapeDtypeStruct((M, N), jnp.bfloat16),
        in_specs=[pl.BlockSpec(memory_space=pltpu.VMEM),
                  pl.BlockSpec(memory_space=pltpu.VMEM)],
        out_specs=pl.BlockSpec(memory_space=pltpu.VMEM),
        scratch_shapes=[
            pltpu.VMEM((M, N), jnp.bfloat16),
            pltpu.VMEM((M, N), jnp.bfloat16),
            pltpu.VMEM((1, M), jnp.int32),
            pltpu.SemaphoreType.DMA((2,)),
            pltpu.SemaphoreType.DMA((2,)),
        ],
        compiler_params=pltpu.CompilerParams(collective_id=0),
    )(x, dest2d)


# baseline (device time: 21035 ns/iter reference)
import functools

import jax
import jax.numpy as jnp
from jax import lax
from jax.experimental import pallas as pl
from jax.experimental.pallas import tpu as pltpu

M = 1024
N = 512


def kernel(x, dest):
    dest2d = dest.reshape(1, M)

    def body(x_ref, dest_ref, out_ref, xsend_ref, xpeer_ref, dpeer_ref,
             send_sems, recv_sems):
        my_x = lax.axis_index("x")
        my_y = lax.axis_index("y")
        peer = (1 - my_x, my_y)

        xsend_ref[...] = x_ref[...].astype(jnp.bfloat16)

        barrier_sem = pltpu.get_barrier_semaphore()
        pl.semaphore_signal(barrier_sem, inc=1, device_id=peer,
                            device_id_type=pl.DeviceIdType.MESH)
        pl.semaphore_wait(barrier_sem, 1)

        rdma_d = pltpu.make_async_remote_copy(
            src_ref=dest_ref, dst_ref=dpeer_ref,
            send_sem=send_sems.at[1], recv_sem=recv_sems.at[1],
            device_id=peer, device_id_type=pl.DeviceIdType.MESH)
        rdma_d.start()
        rdma_x = pltpu.make_async_remote_copy(
            src_ref=xsend_ref, dst_ref=xpeer_ref,
            send_sem=send_sems.at[0], recv_sem=recv_sems.at[0],
            device_id=peer, device_id_type=pl.DeviceIdType.MESH)
        rdma_x.start()
        rdma_d.wait()

        dl = dest_ref[...]
        dp = dpeer_ref[...]
        ml = (dl == my_x)
        mp = (dp == my_x)
        iota_i = lax.broadcasted_iota(jnp.int32, (M, M), 0)
        iota_j = lax.broadcasted_iota(jnp.int32, (M, M), 1)
        tri = (iota_i <= iota_j).astype(jnp.float32)
        csl = jnp.dot(ml.astype(jnp.float32), tri,
                      preferred_element_type=jnp.float32).astype(jnp.int32)
        csp = jnp.dot(mp.astype(jnp.float32), tri,
                      preferred_element_type=jnp.float32).astype(jnp.int32)
        cl = csl[0, M - 1]
        cp = csp[0, M - 1]
        off_l = jnp.where(my_x == 0, 0, cp)
        off_p = jnp.where(my_x == 0, cl, 0)
        posl = off_l + csl - 1
        posp = off_p + csp - 1

        p_l = ((iota_i == posl) & ml).astype(jnp.bfloat16)
        acc = jnp.dot(p_l, xsend_ref[...], preferred_element_type=jnp.float32)

        rdma_x.wait()
        p_p = ((iota_i == posp) & mp).astype(jnp.bfloat16)
        acc = acc + jnp.dot(p_p, xpeer_ref[...],
                            preferred_element_type=jnp.float32)
        out_ref[...] = acc.astype(jnp.bfloat16)

        @functools.partial(pl.run_scoped, sem2=pltpu.SemaphoreType.REGULAR)
        def _(sem2):
            pl.semaphore_signal(sem2, inc=1, device_id=peer,
                                device_id_type=pl.DeviceIdType.MESH)
            pl.semaphore_wait(sem2, 1)

    return pl.pallas_call(
        body,
        out_shape=jax.ShapeDtypeStruct((M, N), jnp.bfloat16),
        in_specs=[pl.BlockSpec(memory_space=pltpu.VMEM),
                  pl.BlockSpec(memory_space=pltpu.VMEM)],
        out_specs=pl.BlockSpec(memory_space=pltpu.VMEM),
        scratch_shapes=[
            pltpu.VMEM((M, N), jnp.bfloat16),
            pltpu.VMEM((M, N), jnp.bfloat16),
            pltpu.VMEM((1, M), jnp.int32),
            pltpu.SemaphoreType.DMA((2,)),
            pltpu.SemaphoreType.DMA((2,)),
        ],
        compiler_params=pltpu.CompilerParams(collective_id=0),
    )(x, dest2d)
